# baseline (speedup 1.0000x reference)
import jax
import jax.numpy as jnp
from jax.experimental import pallas as pl
from jax.experimental.pallas import tpu as pltpu

RADIUS = 0.1
BLOCK_R = 256
SUB_R = 16
SUB_W = 512


def _loss_kernel(s_ref, ss_ref, dT_ref, ds_ref, out_ref, num_acc, cnt_acc):
    i = pl.program_id(0)
    nsteps = pl.num_programs(0)
    f32 = jnp.float32
    r2 = jnp.asarray(RADIUS * RADIUS, f32)
    zero = jnp.zeros((), f32)
    one = jnp.ones((), f32)

    m_total = dT_ref.shape[1]
    n_jt = m_total // SUB_W
    n_rg = BLOCK_R // SUB_R

    s = s_ref[...]
    dT = dT_ref[...]
    sq_s = jnp.sum(s * s, axis=1, keepdims=True)
    sq_d = jnp.sum(dT * dT, axis=0, keepdims=True)
    dotm2 = jnp.dot(s * (-2.0), dT,
                    preferred_element_type=f32)

    dtiles = []
    for jt in range(n_jt):
        js = slice(jt * SUB_W, (jt + 1) * SUB_W)
        fb = jnp.broadcast_to(sq_d[0:1, js], (SUB_R, SUB_W))
        dsb = jnp.broadcast_to(ds_ref[0:1, js], (SUB_R, SUB_W))
        dtiles.append((fb, dsb))

    acc_n = jnp.zeros((SUB_R, SUB_W), f32)
    acc_c = jnp.zeros((SUB_R, SUB_W), f32)
    for rg in range(n_rg):
        rs = slice(rg * SUB_R, (rg + 1) * SUB_R)
        eb = jnp.broadcast_to(sq_s[rs, :], (SUB_R, SUB_W))
        ssb = jnp.broadcast_to(ss_ref[rs, :], (SUB_R, SUB_W))
        for jt in range(n_jt):
            fb, dsb = dtiles[jt]
            js = slice(jt * SUB_W, (jt + 1) * SUB_W)
            d2 = (eb + fb) + dotm2[rs, js]
            m = d2 < r2
            t = jnp.where(m, ssb - dsb, zero)
            acc_n = acc_n + t * t
            acc_c = acc_c + jnp.where(m, one, zero)

    num_step = jnp.sum(acc_n)
    cnt_step = jnp.sum(acc_c)

    @pl.when(i == 0)
    def _init():
        num_acc[0, 0] = num_step
        cnt_acc[0, 0] = cnt_step

    @pl.when(i != 0)
    def _accum():
        num_acc[0, 0] += num_step
        cnt_acc[0, 0] += cnt_step

    @pl.when(i == nsteps - 1)
    def _finish():
        loss = num_acc[0, 0] / jnp.maximum(cnt_acc[0, 0], 1.0)
        out_ref[...] = jnp.full((1, 1), loss, dtype=jnp.float32)


def kernel(src_xyz, src_scores, dst_xyz, dst_scores):
    n = src_xyz.shape[0]
    m = dst_xyz.shape[0]
    ss = src_scores.reshape(n, 1)
    ds = dst_scores.reshape(1, m)
    dT = dst_xyz.T

    grid = (n // BLOCK_R,)
    out = pl.pallas_call(
        _loss_kernel,
        grid=grid,
        in_specs=[
            pl.BlockSpec((BLOCK_R, 3), lambda i: (i, 0)),
            pl.BlockSpec((BLOCK_R, 1), lambda i: (i, 0)),
            pl.BlockSpec((3, m), lambda i: (0, 0)),
            pl.BlockSpec((1, m), lambda i: (0, 0)),
        ],
        out_specs=pl.BlockSpec((1, 1), lambda i: (0, 0)),
        out_shape=jax.ShapeDtypeStruct((1, 1), jnp.float32),
        scratch_shapes=[
            pltpu.SMEM((1, 1), jnp.float32),
            pltpu.SMEM((1, 1), jnp.float32),
        ],
    )(src_xyz, ss, dT, ds)
    return out[0, 0]


# speedup vs baseline: 1.2327x; 1.2327x over previous
import jax
import jax.numpy as jnp
from jax.experimental import pallas as pl
from jax.experimental.pallas import tpu as pltpu

RADIUS = 0.1
BLOCK_R = 512
SUB_R = 16
SUB_W = 512


def _loss_kernel(s_ref, ss_ref, dT_ref, ds_ref, out_ref, num_acc, cnt_acc):
    i = pl.program_id(0)
    nsteps = pl.num_programs(0)
    f32 = jnp.float32
    r2 = jnp.asarray(RADIUS * RADIUS, f32)
    zero = jnp.zeros((), f32)
    one = jnp.ones((), f32)

    m_total = dT_ref.shape[1]
    n_jt = m_total // SUB_W
    n_rg = BLOCK_R // SUB_R

    s = s_ref[...]
    dT = dT_ref[...]
    sq_s = jnp.sum(s * s, axis=1, keepdims=True)
    sq_d = jnp.sum(dT * dT, axis=0, keepdims=True)
    onesc = jnp.ones((s.shape[0], 1), f32)
    onesr = jnp.ones((1, m_total), f32)
    aug_a = jnp.concatenate([s * (-2.0), sq_s, onesc], axis=1)
    aug_b = jnp.concatenate([dT, onesr, sq_d], axis=0)
    d2full = jnp.dot(aug_a, aug_b, preferred_element_type=f32)

    dtiles = []
    for jt in range(n_jt):
        js = slice(jt * SUB_W, (jt + 1) * SUB_W)
        dtiles.append(jnp.broadcast_to(ds_ref[0:1, js], (SUB_R, SUB_W)))

    acc_n = jnp.zeros((SUB_R, SUB_W), f32)
    acc_c = jnp.zeros((SUB_R, SUB_W), f32)
    for rg in range(n_rg):
        rs = slice(rg * SUB_R, (rg + 1) * SUB_R)
        ssb = jnp.broadcast_to(ss_ref[rs, :], (SUB_R, SUB_W))
        for jt in range(n_jt):
            dsb = dtiles[jt]
            js = slice(jt * SUB_W, (jt + 1) * SUB_W)
            m = d2full[rs, js] < r2
            t = jnp.where(m, ssb - dsb, zero)
            acc_n = acc_n + t * t
            acc_c = acc_c + jnp.where(m, one, zero)

    num_step = jnp.sum(acc_n)
    cnt_step = jnp.sum(acc_c)

    @pl.when(i == 0)
    def _init():
        num_acc[0, 0] = num_step
        cnt_acc[0, 0] = cnt_step

    @pl.when(i != 0)
    def _accum():
        num_acc[0, 0] += num_step
        cnt_acc[0, 0] += cnt_step

    @pl.when(i == nsteps - 1)
    def _finish():
        loss = num_acc[0, 0] / jnp.maximum(cnt_acc[0, 0], 1.0)
        out_ref[...] = jnp.full((1, 1), loss, dtype=jnp.float32)


def kernel(src_xyz, src_scores, dst_xyz, dst_scores):
    n = src_xyz.shape[0]
    m = dst_xyz.shape[0]
    ss = src_scores.reshape(n, 1)
    ds = dst_scores.reshape(1, m)
    dT = dst_xyz.T

    grid = (n // BLOCK_R,)
    out = pl.pallas_call(
        _loss_kernel,
        grid=grid,
        in_specs=[
            pl.BlockSpec((BLOCK_R, 3), lambda i: (i, 0)),
            pl.BlockSpec((BLOCK_R, 1), lambda i: (i, 0)),
            pl.BlockSpec((3, m), lambda i: (0, 0)),
            pl.BlockSpec((1, m), lambda i: (0, 0)),
        ],
        out_specs=pl.BlockSpec((1, 1), lambda i: (0, 0)),
        out_shape=jax.ShapeDtypeStruct((1, 1), jnp.float32),
        scratch_shapes=[
            pltpu.SMEM((1, 1), jnp.float32),
            pltpu.SMEM((1, 1), jnp.float32),
        ],
    )(src_xyz, ss, dT, ds)
    return out[0, 0]
